# baseline (speedup 1.0000x reference)
import jax
import jax.numpy as jnp
from jax import lax
from jax.experimental import pallas as pl
from jax.experimental.pallas import tpu as pltpu

_RATIO = 0.7
_H = 512
_W = 512


def _fused_body(mask_hbm, img_ref, out_ref, bbox, mvm, sem):
    @pl.when(pl.program_id(0) == 0)
    def _():
        cp = pltpu.make_async_copy(mask_hbm, mvm, sem)
        cp.start()
        cp.wait()
        m = mvm[...]
        nz = m != 0.0
        row_id = lax.broadcasted_iota(jnp.int32, (_H, _W), 0)
        col_id = lax.broadcasted_iota(jnp.int32, (_H, _W), 1)
        top = jnp.min(jnp.where(nz, row_id, _H))
        bottom = jnp.max(jnp.where(nz, row_id, -1))
        left = jnp.min(jnp.where(nz, col_id, _W))
        right = jnp.max(jnp.where(nz, col_id, -1))
        bbox[0] = jnp.floor(top * _RATIO).astype(jnp.int32)
        bbox[1] = jnp.floor(left * _RATIO).astype(jnp.int32)
        bbox[2] = jnp.floor(bottom + (_H - bottom) * (1.0 - _RATIO)).astype(jnp.int32)
        bbox[3] = jnp.floor(right + (_W - right) * (1.0 - _RATIO)).astype(jnp.int32)

    t = bbox[0]
    l = bbox[1]
    b = bbox[2]
    r = bbox[3]
    row_id = lax.broadcasted_iota(jnp.int32, (_H, _W), 0)
    col_id = lax.broadcasted_iota(jnp.int32, (_H, _W), 1)
    region = (row_id >= t) & (row_id < b) & (col_id >= l) & (col_id < r)
    out_ref[...] = jnp.where(region[None, :, :], img_ref[...], -1.0)


@jax.jit
def kernel(image, cover, mask):
    del cover
    m = mask[0, 0]
    n = image.shape[0] * image.shape[1]
    x = image.reshape(n, _H, _W)
    blk = 12
    out = pl.pallas_call(
        _fused_body,
        grid=(n // blk,),
        in_specs=[
            pl.BlockSpec(memory_space=pl.ANY),
            pl.BlockSpec((blk, _H, _W), lambda i: (i, 0, 0)),
        ],
        out_specs=pl.BlockSpec((blk, _H, _W), lambda i: (i, 0, 0)),
        out_shape=jax.ShapeDtypeStruct((n, _H, _W), jnp.float32),
        scratch_shapes=[
            pltpu.SMEM((4,), jnp.int32),
            pltpu.VMEM((_H, _W), jnp.float32),
            pltpu.SemaphoreType.DMA,
        ],
        compiler_params=pltpu.CompilerParams(
            dimension_semantics=("arbitrary",),
        ),
    )(m, x)
    return out.reshape(image.shape)
